# Initial kernel scaffold; baseline (speedup 1.0000x reference)
import jax
import jax.numpy as jnp
from jax.experimental import pallas as pl


def kernel(points, queries, feat_prev, bn1_g, bn1_b, W1, b1, bn2_g, bn2_b, W2, b2, bn3_g, bn3_b, convW, convb):
    raise NotImplementedError("write your pallas kernel here")


# speedup vs baseline: 8.9247x; 8.9247x over previous
import functools

import jax
import jax.numpy as jnp
import numpy as np
from jax import lax
from jax.experimental import pallas as pl
from jax.experimental.pallas import tpu as pltpu
from jax.experimental.pallas import tpu_sc as plsc

_B, _L, _N, _M, _K, _DIV, _FPREV, _PF, _OUT = 4, 1, 2048, 1024, 32, 4, 128, 64, 256
_S = _K // _DIV
_C3 = _PF + _FPREV
_EPS = 1e-5
_BM = _B * _M
_ROWS = _BM * _K
_NW = 32
_RT = _ROWS // _NW
_QT = _BM // _NW
_LN = 16


def _knn_topk(q2, ptsT, *, interpret=False):
    MBLK = 256
    INF = np.float32(np.inf)

    def body(q_ref, p_ref, out_ref, d2_ref):
        b = pl.program_id(0)
        qr = q_ref[0]
        pr = p_ref[0]
        qx, qy, qz = qr[:, 0:1], qr[:, 1:2], qr[:, 2:3]
        px, py, pz = pr[0:1, :], pr[1:2, :], pr[2:3, :]
        dx = qx - px
        dy = qy - py
        dz = qz - pz
        d2_ref[...] = (dx * dx + dy * dy) + dz * dz
        iota = lax.broadcasted_iota(jnp.int32, (MBLK, _N), 1)
        off = b * _N
        for k in range(_K):
            d2 = d2_ref[...]
            m = jnp.min(d2, axis=1, keepdims=True)
            ci = jnp.min(jnp.where(d2 == m, iota, _N), axis=1, keepdims=True)
            out_ref[0, :, k:k + 1] = ci + off
            d2_ref[...] = jnp.where(iota == ci, INF, d2)

    return pl.pallas_call(
        body,
        grid=(_B, _M // MBLK),
        in_specs=[
            pl.BlockSpec((1, MBLK, 3), lambda b, i: (b, i, 0)),
            pl.BlockSpec((1, 3, _N), lambda b, i: (b, 0, 0)),
        ],
        out_specs=pl.BlockSpec((1, MBLK, _K), lambda b, i: (b, i, 0)),
        out_shape=jax.ShapeDtypeStruct((_B, _M, _K), jnp.int32),
        scratch_shapes=[pltpu.VMEM((MBLK, _N), jnp.float32)],
        interpret=interpret,
    )(q2, ptsT)


def _sc_gather(px, py, pz, qxr, qyr, qzr, idx_flat, idx2d, fp2d):
    mesh = plsc.VectorSubcoreMesh(core_axis_name="c", subcore_axis_name="s")
    NCH1 = 256
    NJ = _RT // 128
    WPJ = 128 // _S

    @functools.partial(
        pl.kernel,
        mesh=mesh,
        compiler_params=pltpu.CompilerParams(needs_layout_passes=False),
        out_type=[
            jax.ShapeDtypeStruct((_ROWS,), jnp.float32),
            jax.ShapeDtypeStruct((_ROWS,), jnp.float32),
            jax.ShapeDtypeStruct((_ROWS,), jnp.float32),
            jax.ShapeDtypeStruct((_BM * _DIV, _FPREV), jnp.float32),
            jax.ShapeDtypeStruct((_NW, 128), jnp.float32),
        ],
        scratch_types=[
            pltpu.VMEM((_B * _N,), jnp.float32),
            pltpu.VMEM((_B * _N,), jnp.float32),
            pltpu.VMEM((_B * _N,), jnp.float32),
            pltpu.VMEM((_RT,), jnp.float32),
            pltpu.VMEM((_RT,), jnp.float32),
            pltpu.VMEM((_RT,), jnp.float32),
            pltpu.VMEM((_RT,), jnp.int32),
            pltpu.VMEM((_RT // 128, 128), jnp.int32),
            pltpu.VMEM((_RT,), jnp.float32),
            pltpu.VMEM((_RT,), jnp.float32),
            pltpu.VMEM((_RT,), jnp.float32),
            pltpu.VMEM((128, _FPREV), jnp.float32),
            pltpu.VMEM((WPJ, _FPREV), jnp.float32),
            pltpu.VMEM((128,), jnp.float32),
            pltpu.VMEM((16,), jnp.float32),
            pltpu.VMEM((16,), jnp.float32),
            pltpu.VMEM((16,), jnp.float32),
            pltpu.VMEM((16,), jnp.float32),
            pltpu.VMEM((16,), jnp.float32),
            pltpu.VMEM((16,), jnp.float32),
            pltpu.SemaphoreType.DMA,
        ],
    )
    def body(px_h, py_h, pz_h, qx_h, qy_h, qz_h, idxf_h, idx2_h, fp_h,
             lx_h, ly_h, lz_h, fmax_h, st1_h,
             pxv, pyv, pzv, qxv, qyv, qzv, idxf_v, idxv,
             lxv, lyv, lzv, gbuf, fwin, st1v,
             sx, sy, sz, sxx, syy, szz, sem):
        wid = lax.axis_index("s") * 2 + lax.axis_index("c")
        rows_base = pl.multiple_of(wid * _RT, _RT)
        irow_base = pl.multiple_of(wid * (_RT // 128), _RT // 128)

        pltpu.sync_copy(px_h, pxv)
        pltpu.sync_copy(py_h, pyv)
        pltpu.sync_copy(pz_h, pzv)
        pltpu.sync_copy(qx_h.at[pl.ds(rows_base, _RT)], qxv)
        pltpu.sync_copy(qy_h.at[pl.ds(rows_base, _RT)], qyv)
        pltpu.sync_copy(qz_h.at[pl.ds(rows_base, _RT)], qzv)
        pltpu.sync_copy(idxf_h.at[pl.ds(rows_base, _RT)], idxf_v)
        pltpu.sync_copy(idx2_h.at[pl.ds(irow_base, _RT // 128)], idxv)

        zero = jnp.zeros((_LN,), jnp.float32)
        sx[...] = zero
        sy[...] = zero
        sz[...] = zero
        sxx[...] = zero
        syy[...] = zero
        szz[...] = zero

        def p1(i, carry):
            o = pl.multiple_of(i * _LN, _LN)
            iv = idxf_v[pl.ds(o, _LN)]
            gx = plsc.load_gather(pxv, [iv])
            gy = plsc.load_gather(pyv, [iv])
            gz = plsc.load_gather(pzv, [iv])
            lx = qxv[pl.ds(o, _LN)] - gx
            ly = qyv[pl.ds(o, _LN)] - gy
            lz = qzv[pl.ds(o, _LN)] - gz
            lxv[pl.ds(o, _LN)] = lx
            lyv[pl.ds(o, _LN)] = ly
            lzv[pl.ds(o, _LN)] = lz
            sx[...] = sx[...] + lx
            sy[...] = sy[...] + ly
            sz[...] = sz[...] + lz
            sxx[...] = sxx[...] + lx * lx
            syy[...] = syy[...] + ly * ly
            szz[...] = szz[...] + lz * lz
            return carry

        lax.fori_loop(0, NCH1, p1, 0)

        pltpu.sync_copy(lxv, lx_h.at[pl.ds(rows_base, _RT)])
        pltpu.sync_copy(lyv, ly_h.at[pl.ds(rows_base, _RT)])
        pltpu.sync_copy(lzv, lz_h.at[pl.ds(rows_base, _RT)])

        st1v[pl.ds(0, 16)] = sx[...]
        st1v[pl.ds(16, 16)] = sy[...]
        st1v[pl.ds(32, 16)] = sz[...]
        st1v[pl.ds(48, 16)] = sxx[...]
        st1v[pl.ds(64, 16)] = syy[...]
        st1v[pl.ds(80, 16)] = szz[...]
        st1v[pl.ds(96, 16)] = zero
        st1v[pl.ds(112, 16)] = zero
        pltpu.sync_copy(st1v, st1_h.at[wid])

        def p2(j, carry):
            pltpu.async_copy(fp_h.at[idxv.at[j]], gbuf, sem).wait()
            for w in range(WPJ):
                for c in range(_FPREV // _LN):
                    m = gbuf[w * _S, pl.ds(c * _LN, _LN)]
                    for r in range(1, _S):
                        m = jnp.maximum(m, gbuf[w * _S + r, pl.ds(c * _LN, _LN)])
                    fwin[w, pl.ds(c * _LN, _LN)] = m
            wrow = pl.multiple_of(wid * (_QT * _DIV) + j * WPJ, WPJ)
            pltpu.sync_copy(fwin, fmax_h.at[pl.ds(wrow, WPJ)])
            return carry

        lax.fori_loop(0, NJ, p2, 0)

    return body(px, py, pz, qxr, qyr, qzr, idx_flat, idx2d, fp2d)


def _mlp1(X, W1effT, b1eff, *, interpret=False):
    RB = 8192
    nblk = _ROWS // RB

    def body(x_ref, w_ref, b_ref, r_ref, p_ref):
        x = x_ref[...]
        r1 = jnp.maximum(
            jnp.dot(x, w_ref[...], preferred_element_type=jnp.float32)
            + b_ref[...], 0.0)
        r_ref[...] = r1
        p_ref[0, 0:1, :] = jnp.sum(r1, axis=0, keepdims=True)
        p_ref[0, 1:2, :] = jnp.sum(r1 * r1, axis=0, keepdims=True)

    return pl.pallas_call(
        body,
        grid=(nblk,),
        in_specs=[
            pl.BlockSpec((RB, 3), lambda i: (i, 0)),
            pl.BlockSpec((3, _PF // 2), lambda i: (0, 0)),
            pl.BlockSpec((1, _PF // 2), lambda i: (0, 0)),
        ],
        out_specs=[
            pl.BlockSpec((RB, _PF // 2), lambda i: (i, 0)),
            pl.BlockSpec((1, 2, _PF // 2), lambda i: (i, 0, 0)),
        ],
        out_shape=[
            jax.ShapeDtypeStruct((_ROWS, _PF // 2), jnp.float32),
            jax.ShapeDtypeStruct((nblk, 2, _PF // 2), jnp.float32),
        ],
        interpret=interpret,
    )(X, W1effT, b1eff)


def _mlp2pool(r1, fmax, W2effT, b2eff, *, interpret=False):
    RB = 8192
    nblk = _ROWS // RB
    WB = RB // _S

    def body(r_ref, f_ref, w_ref, b_ref, hm_ref, ph_ref, pf_ref):
        h = jnp.maximum(
            jnp.dot(r_ref[...], w_ref[...], preferred_element_type=jnp.float32)
            + b_ref[...], 0.0)
        hm = jnp.max(h.reshape(WB, _S, _PF), axis=1)
        hm_ref[...] = hm
        ph_ref[0, 0:1, :] = jnp.sum(hm, axis=0, keepdims=True)
        ph_ref[0, 1:2, :] = jnp.sum(hm * hm, axis=0, keepdims=True)
        f = f_ref[...]
        pf_ref[0, 0:1, :] = jnp.sum(f, axis=0, keepdims=True)
        pf_ref[0, 1:2, :] = jnp.sum(f * f, axis=0, keepdims=True)

    return pl.pallas_call(
        body,
        grid=(nblk,),
        in_specs=[
            pl.BlockSpec((RB, _PF // 2), lambda i: (i, 0)),
            pl.BlockSpec((WB, _FPREV), lambda i: (i, 0)),
            pl.BlockSpec((_PF // 2, _PF), lambda i: (0, 0)),
            pl.BlockSpec((1, _PF), lambda i: (0, 0)),
        ],
        out_specs=[
            pl.BlockSpec((WB, _PF), lambda i: (i, 0)),
            pl.BlockSpec((1, 2, _PF), lambda i: (i, 0, 0)),
            pl.BlockSpec((1, 2, _FPREV), lambda i: (i, 0, 0)),
        ],
        out_shape=[
            jax.ShapeDtypeStruct((_BM * _DIV, _PF), jnp.float32),
            jax.ShapeDtypeStruct((nblk, 2, _PF), jnp.float32),
            jax.ShapeDtypeStruct((nblk, 2, _FPREV), jnp.float32),
        ],
        interpret=interpret,
    )(r1, fmax, W2effT, b2eff)


def _conv(hm2, fm2, Wh, Wf, sh, th, sf, tf, cb, *, interpret=False):
    RB = 512
    nblk = _BM // RB
    CH = _DIV * _PF
    CF = _DIV * _FPREV

    def body(h_ref, f_ref, wh_ref, wf_ref, sh_ref, th_ref, sf_ref, tf_ref,
             cb_ref, o_ref):
        a = h_ref[...] * sh_ref[...] + th_ref[...]
        b = f_ref[...] * sf_ref[...] + tf_ref[...]
        o_ref[...] = (
            jnp.dot(a, wh_ref[...], preferred_element_type=jnp.float32)
            + jnp.dot(b, wf_ref[...], preferred_element_type=jnp.float32)
            + cb_ref[...])

    return pl.pallas_call(
        body,
        grid=(nblk,),
        in_specs=[
            pl.BlockSpec((RB, CH), lambda i: (i, 0)),
            pl.BlockSpec((RB, CF), lambda i: (i, 0)),
            pl.BlockSpec((CH, _OUT), lambda i: (0, 0)),
            pl.BlockSpec((CF, _OUT), lambda i: (0, 0)),
            pl.BlockSpec((1, CH), lambda i: (0, 0)),
            pl.BlockSpec((1, CH), lambda i: (0, 0)),
            pl.BlockSpec((1, CF), lambda i: (0, 0)),
            pl.BlockSpec((1, CF), lambda i: (0, 0)),
            pl.BlockSpec((1, _OUT), lambda i: (0, 0)),
        ],
        out_specs=pl.BlockSpec((RB, _OUT), lambda i: (i, 0)),
        out_shape=jax.ShapeDtypeStruct((_BM, _OUT), jnp.float32),
        interpret=interpret,
    )(hm2, fm2, Wh, Wf, sh, th, sf, tf, cb)


def kernel(points, queries, feat_prev, bn1_g, bn1_b, W1, b1, bn2_g, bn2_b,
           W2, b2, bn3_g, bn3_b, convW, convb):
    f32 = jnp.float32
    pts = points.reshape(_B, _N, 3)
    q2 = queries.reshape(_B, _M, 3)
    fp2d = feat_prev.reshape(_B * _N, _FPREV)
    ptsT = pts.transpose(0, 2, 1)

    idxf = _knn_topk(q2, ptsT)
    idx_flat = idxf.reshape(_ROWS)
    idx2d = idxf.reshape(_ROWS // 128, 128)

    pf3 = pts.reshape(_B * _N, 3)
    px, py, pz = pf3[:, 0], pf3[:, 1], pf3[:, 2]
    qf3 = q2.reshape(_BM, 3)
    qxr = jnp.repeat(qf3[:, 0], _K)
    qyr = jnp.repeat(qf3[:, 1], _K)
    qzr = jnp.repeat(qf3[:, 2], _K)
    lx, ly, lz, fmax, st1 = _sc_gather(px, py, pz, qxr, qyr, qzr,
                                       idx_flat, idx2d, fp2d)

    cnt1 = f32(_ROWS)
    st1r = st1.reshape(_NW, 8, _LN)
    s1sum = jnp.sum(st1r[:, 0:3, :], axis=(0, 2))
    s1sq = jnp.sum(st1r[:, 3:6, :], axis=(0, 2))
    mu1 = s1sum / cnt1
    var1 = s1sq / cnt1 - mu1 * mu1
    inv1 = lax.rsqrt(var1 + _EPS)
    sc1 = bn1_g * inv1
    sh1 = bn1_b - mu1 * sc1
    W1eff = W1 * sc1[None, :]
    b1eff = b1 + W1 @ sh1

    X = jnp.stack([lx, ly, lz], axis=-1)
    r1, p3 = _mlp1(X, W1eff.T, b1eff.reshape(1, _PF // 2))

    mu2 = jnp.sum(p3[:, 0, :], axis=0) / cnt1
    var2 = jnp.sum(p3[:, 1, :], axis=0) / cnt1 - mu2 * mu2
    inv2 = lax.rsqrt(var2 + _EPS)
    sc2 = bn2_g * inv2
    sh2 = bn2_b - mu2 * sc2
    W2eff = W2 * sc2[None, :]
    b2eff = b2 + W2 @ sh2

    hmax, ph, pf = _mlp2pool(r1, fmax, W2eff.T, b2eff.reshape(1, _PF))

    cnt3 = f32(_BM * _DIV)
    mu3h = jnp.sum(ph[:, 0, :], axis=0) / cnt3
    var3h = jnp.sum(ph[:, 1, :], axis=0) / cnt3 - mu3h * mu3h
    mu3f = jnp.sum(pf[:, 0, :], axis=0) / cnt3
    var3f = jnp.sum(pf[:, 1, :], axis=0) / cnt3 - mu3f * mu3f
    inv3h = lax.rsqrt(var3h + _EPS)
    inv3f = lax.rsqrt(var3f + _EPS)
    sch = bn3_g[:_PF] * inv3h
    thv = bn3_b[:_PF] - mu3h * sch
    scf = bn3_g[_PF:] * inv3f
    tfv = bn3_b[_PF:] - mu3f * scf

    Wh = convW[:, :_PF, :].transpose(2, 1, 0).reshape(_DIV * _PF, _OUT)
    Wf = convW[:, _PF:, :].transpose(2, 1, 0).reshape(_DIV * _FPREV, _OUT)
    sh256 = jnp.tile(sch, _DIV).reshape(1, _DIV * _PF)
    th256 = jnp.tile(thv, _DIV).reshape(1, _DIV * _PF)
    sf512 = jnp.tile(scf, _DIV).reshape(1, _DIV * _FPREV)
    tf512 = jnp.tile(tfv, _DIV).reshape(1, _DIV * _FPREV)
    hm2 = hmax.reshape(_BM, _DIV * _PF)
    fm2 = fmax.reshape(_BM, _DIV * _FPREV)
    out = _conv(hm2, fm2, Wh, Wf, sh256, th256, sf512, tf512,
                convb.reshape(1, _OUT))
    return out.reshape(_B, _L, _M, _OUT)


# speedup vs baseline: 10.7488x; 1.2044x over previous
import functools

import jax
import jax.numpy as jnp
import numpy as np
from jax import lax
from jax.experimental import pallas as pl
from jax.experimental.pallas import tpu as pltpu
from jax.experimental.pallas import tpu_sc as plsc

_B, _L, _N, _M, _K, _DIV, _FPREV, _PF, _OUT = 4, 1, 2048, 1024, 32, 4, 128, 64, 256
_S = _K // _DIV
_C3 = _PF + _FPREV
_EPS = 1e-5
_BM = _B * _M
_ROWS = _BM * _K
_NW = 32
_RT = _ROWS // _NW
_QT = _BM // _NW
_LN = 16


def _knn_topk(q2, ptsT, *, interpret=False):
    MBLK = 256
    INF = np.float32(np.inf)

    def body(q_ref, p_ref, out_ref, d2_ref):
        b = pl.program_id(0)
        qr = q_ref[0]
        pr = p_ref[0]
        qx, qy, qz = qr[:, 0:1], qr[:, 1:2], qr[:, 2:3]
        px, py, pz = pr[0:1, :], pr[1:2, :], pr[2:3, :]
        dx = qx - px
        dy = qy - py
        dz = qz - pz
        d2_ref[...] = (dx * dx + dy * dy) + dz * dz
        iota = lax.broadcasted_iota(jnp.int32, (MBLK, _N), 1).astype(jnp.float32)
        off = b * _N
        NF = np.float32(_N)
        for k in range(_K):
            d2 = d2_ref[...]
            m = jnp.min(d2, axis=1, keepdims=True)
            ci = jnp.min(jnp.where(d2 == m, iota, NF), axis=1, keepdims=True)
            out_ref[0, :, k:k + 1] = ci.astype(jnp.int32) + off
            d2_ref[...] = jnp.where(iota == ci, INF, d2)

    return pl.pallas_call(
        body,
        grid=(_B, _M // MBLK),
        in_specs=[
            pl.BlockSpec((1, MBLK, 3), lambda b, i: (b, i, 0)),
            pl.BlockSpec((1, 3, _N), lambda b, i: (b, 0, 0)),
        ],
        out_specs=pl.BlockSpec((1, MBLK, _K), lambda b, i: (b, i, 0)),
        out_shape=jax.ShapeDtypeStruct((_B, _M, _K), jnp.int32),
        scratch_shapes=[pltpu.VMEM((MBLK, _N), jnp.float32)],
        interpret=interpret,
    )(q2, ptsT)


def _sc_gather(px, py, pz, qxr, qyr, qzr, idx_flat, idx2d, fp2d):
    mesh = plsc.VectorSubcoreMesh(core_axis_name="c", subcore_axis_name="s")
    NCH1 = 256
    NJ = _RT // 128
    WPJ = 128 // _S

    @functools.partial(
        pl.kernel,
        mesh=mesh,
        compiler_params=pltpu.CompilerParams(needs_layout_passes=False),
        out_type=[
            jax.ShapeDtypeStruct((_ROWS,), jnp.float32),
            jax.ShapeDtypeStruct((_ROWS,), jnp.float32),
            jax.ShapeDtypeStruct((_ROWS,), jnp.float32),
            jax.ShapeDtypeStruct((_BM * _DIV, _FPREV), jnp.float32),
            jax.ShapeDtypeStruct((_NW, 128), jnp.float32),
        ],
        scratch_types=[
            pltpu.VMEM((_B * _N,), jnp.float32),
            pltpu.VMEM((_B * _N,), jnp.float32),
            pltpu.VMEM((_B * _N,), jnp.float32),
            pltpu.VMEM((_RT,), jnp.float32),
            pltpu.VMEM((_RT,), jnp.float32),
            pltpu.VMEM((_RT,), jnp.float32),
            pltpu.VMEM((_RT,), jnp.int32),
            pltpu.VMEM((_RT // 128, 128), jnp.int32),
            pltpu.VMEM((_RT,), jnp.float32),
            pltpu.VMEM((_RT,), jnp.float32),
            pltpu.VMEM((_RT,), jnp.float32),
            pltpu.VMEM((128, _FPREV), jnp.float32),
            pltpu.VMEM((128, _FPREV), jnp.float32),
            pltpu.VMEM((WPJ, _FPREV), jnp.float32),
            pltpu.VMEM((WPJ, _FPREV), jnp.float32),
            pltpu.VMEM((128,), jnp.float32),
            pltpu.VMEM((16,), jnp.float32),
            pltpu.VMEM((16,), jnp.float32),
            pltpu.VMEM((16,), jnp.float32),
            pltpu.VMEM((16,), jnp.float32),
            pltpu.VMEM((16,), jnp.float32),
            pltpu.VMEM((16,), jnp.float32),
            pltpu.SemaphoreType.DMA,
            pltpu.SemaphoreType.DMA,
            pltpu.SemaphoreType.DMA,
            pltpu.SemaphoreType.DMA,
            pltpu.SemaphoreType.DMA,
        ],
    )
    def body(px_h, py_h, pz_h, qx_h, qy_h, qz_h, idxf_h, idx2_h, fp_h,
             lx_h, ly_h, lz_h, fmax_h, st1_h,
             pxv, pyv, pzv, qxv, qyv, qzv, idxf_v, idxv,
             lxv, lyv, lzv, gbuf0, gbuf1, fwin0, fwin1, st1v,
             sx, sy, sz, sxx, syy, szz, sem, gsem0, gsem1, wsem0, wsem1):
        wid = lax.axis_index("s") * 2 + lax.axis_index("c")
        rows_base = pl.multiple_of(wid * _RT, _RT)
        irow_base = pl.multiple_of(wid * (_RT // 128), _RT // 128)

        cps = [
            pltpu.async_copy(px_h, pxv, sem),
            pltpu.async_copy(py_h, pyv, sem),
            pltpu.async_copy(pz_h, pzv, sem),
            pltpu.async_copy(qx_h.at[pl.ds(rows_base, _RT)], qxv, sem),
            pltpu.async_copy(qy_h.at[pl.ds(rows_base, _RT)], qyv, sem),
            pltpu.async_copy(qz_h.at[pl.ds(rows_base, _RT)], qzv, sem),
            pltpu.async_copy(idxf_h.at[pl.ds(rows_base, _RT)], idxf_v, sem),
            pltpu.async_copy(idx2_h.at[pl.ds(irow_base, _RT // 128)], idxv, sem),
        ]
        for cp in cps:
            cp.wait()

        zero = jnp.zeros((_LN,), jnp.float32)
        sx[...] = zero
        sy[...] = zero
        sz[...] = zero
        sxx[...] = zero
        syy[...] = zero
        szz[...] = zero

        def p1(i, carry):
            o = pl.multiple_of(i * _LN, _LN)
            iv = idxf_v[pl.ds(o, _LN)]
            gx = plsc.load_gather(pxv, [iv])
            gy = plsc.load_gather(pyv, [iv])
            gz = plsc.load_gather(pzv, [iv])
            lx = qxv[pl.ds(o, _LN)] - gx
            ly = qyv[pl.ds(o, _LN)] - gy
            lz = qzv[pl.ds(o, _LN)] - gz
            lxv[pl.ds(o, _LN)] = lx
            lyv[pl.ds(o, _LN)] = ly
            lzv[pl.ds(o, _LN)] = lz
            sx[...] = sx[...] + lx
            sy[...] = sy[...] + ly
            sz[...] = sz[...] + lz
            sxx[...] = sxx[...] + lx * lx
            syy[...] = syy[...] + ly * ly
            szz[...] = szz[...] + lz * lz
            return carry

        lax.fori_loop(0, NCH1, p1, 0)

        lcps = [
            pltpu.async_copy(lxv, lx_h.at[pl.ds(rows_base, _RT)], sem),
            pltpu.async_copy(lyv, ly_h.at[pl.ds(rows_base, _RT)], sem),
            pltpu.async_copy(lzv, lz_h.at[pl.ds(rows_base, _RT)], sem),
        ]

        st1v[pl.ds(0, 16)] = sx[...]
        st1v[pl.ds(16, 16)] = sy[...]
        st1v[pl.ds(32, 16)] = sz[...]
        st1v[pl.ds(48, 16)] = sxx[...]
        st1v[pl.ds(64, 16)] = syy[...]
        st1v[pl.ds(80, 16)] = szz[...]
        st1v[pl.ds(96, 16)] = zero
        st1v[pl.ds(112, 16)] = zero
        lcps.append(pltpu.async_copy(st1v, st1_h.at[wid], sem))

        wrow0 = pl.multiple_of(wid * (_QT * _DIV), WPJ)

        def pool(gb, fw):
            for w in range(WPJ):
                for c in range(_FPREV // _LN):
                    m = gb[w * _S, pl.ds(c * _LN, _LN)]
                    for r in range(1, _S):
                        m = jnp.maximum(m, gb[w * _S + r, pl.ds(c * _LN, _LN)])
                    fw[w, pl.ds(c * _LN, _LN)] = m

        pltpu.async_copy(fp_h.at[idxv.at[0]], gbuf0, gsem0)

        def p2(i, carry):
            j = i * 2
            pltpu.make_async_copy(fp_h.at[idxv.at[j]], gbuf0, gsem0).wait()
            pltpu.async_copy(fp_h.at[idxv.at[j + 1]], gbuf1, gsem1)

            @pl.when(i > 0)
            def _():
                pltpu.make_async_copy(
                    fwin0, fmax_h.at[pl.ds(wrow0, WPJ)], wsem0).wait()

            pool(gbuf0, fwin0)
            pltpu.async_copy(fwin0, fmax_h.at[pl.ds(wrow0 + j * WPJ, WPJ)],
                             wsem0)

            pltpu.make_async_copy(fp_h.at[idxv.at[j + 1]], gbuf1, gsem1).wait()

            @pl.when(i < (NJ // 2 - 1))
            def _():
                pltpu.async_copy(fp_h.at[idxv.at[j + 2]], gbuf0, gsem0)

            @pl.when(i > 0)
            def _():
                pltpu.make_async_copy(
                    fwin1, fmax_h.at[pl.ds(wrow0, WPJ)], wsem1).wait()

            pool(gbuf1, fwin1)
            pltpu.async_copy(fwin1, fmax_h.at[pl.ds(wrow0 + (j + 1) * WPJ, WPJ)],
                             wsem1)
            return carry

        lax.fori_loop(0, NJ // 2, p2, 0)
        pltpu.make_async_copy(fwin0, fmax_h.at[pl.ds(wrow0, WPJ)], wsem0).wait()
        pltpu.make_async_copy(fwin1, fmax_h.at[pl.ds(wrow0, WPJ)], wsem1).wait()
        for cp in lcps:
            cp.wait()

    return body(px, py, pz, qxr, qyr, qzr, idx_flat, idx2d, fp2d)


def _mlp1(X, W1effT, b1eff, *, interpret=False):
    RB = 8192
    nblk = _ROWS // RB

    def body(x_ref, w_ref, b_ref, r_ref, p_ref):
        x = x_ref[...]
        r1 = jnp.maximum(
            jnp.dot(x, w_ref[...], preferred_element_type=jnp.float32)
            + b_ref[...], 0.0)
        r_ref[...] = r1
        p_ref[0, 0:1, :] = jnp.sum(r1, axis=0, keepdims=True)
        p_ref[0, 1:2, :] = jnp.sum(r1 * r1, axis=0, keepdims=True)

    return pl.pallas_call(
        body,
        grid=(nblk,),
        in_specs=[
            pl.BlockSpec((RB, 3), lambda i: (i, 0)),
            pl.BlockSpec((3, _PF // 2), lambda i: (0, 0)),
            pl.BlockSpec((1, _PF // 2), lambda i: (0, 0)),
        ],
        out_specs=[
            pl.BlockSpec((RB, _PF // 2), lambda i: (i, 0)),
            pl.BlockSpec((1, 2, _PF // 2), lambda i: (i, 0, 0)),
        ],
        out_shape=[
            jax.ShapeDtypeStruct((_ROWS, _PF // 2), jnp.float32),
            jax.ShapeDtypeStruct((nblk, 2, _PF // 2), jnp.float32),
        ],
        interpret=interpret,
    )(X, W1effT, b1eff)


def _mlp2pool(r1, fmax, W2effT, b2eff, *, interpret=False):
    RB = 8192
    nblk = _ROWS // RB
    WB = RB // _S

    def body(r_ref, f_ref, w_ref, b_ref, hm_ref, ph_ref, pf_ref):
        h = jnp.maximum(
            jnp.dot(r_ref[...], w_ref[...], preferred_element_type=jnp.float32)
            + b_ref[...], 0.0)
        hm = jnp.max(h.reshape(WB, _S, _PF), axis=1)
        hm_ref[...] = hm
        ph_ref[0, 0:1, :] = jnp.sum(hm, axis=0, keepdims=True)
        ph_ref[0, 1:2, :] = jnp.sum(hm * hm, axis=0, keepdims=True)
        f = f_ref[...]
        pf_ref[0, 0:1, :] = jnp.sum(f, axis=0, keepdims=True)
        pf_ref[0, 1:2, :] = jnp.sum(f * f, axis=0, keepdims=True)

    return pl.pallas_call(
        body,
        grid=(nblk,),
        in_specs=[
            pl.BlockSpec((RB, _PF // 2), lambda i: (i, 0)),
            pl.BlockSpec((WB, _FPREV), lambda i: (i, 0)),
            pl.BlockSpec((_PF // 2, _PF), lambda i: (0, 0)),
            pl.BlockSpec((1, _PF), lambda i: (0, 0)),
        ],
        out_specs=[
            pl.BlockSpec((WB, _PF), lambda i: (i, 0)),
            pl.BlockSpec((1, 2, _PF), lambda i: (i, 0, 0)),
            pl.BlockSpec((1, 2, _FPREV), lambda i: (i, 0, 0)),
        ],
        out_shape=[
            jax.ShapeDtypeStruct((_BM * _DIV, _PF), jnp.float32),
            jax.ShapeDtypeStruct((nblk, 2, _PF), jnp.float32),
            jax.ShapeDtypeStruct((nblk, 2, _FPREV), jnp.float32),
        ],
        interpret=interpret,
    )(r1, fmax, W2effT, b2eff)


def _conv(hm2, fm2, Wh, Wf, sh, th, sf, tf, cb, *, interpret=False):
    RB = 512
    nblk = _BM // RB
    CH = _DIV * _PF
    CF = _DIV * _FPREV

    def body(h_ref, f_ref, wh_ref, wf_ref, sh_ref, th_ref, sf_ref, tf_ref,
             cb_ref, o_ref):
        a = h_ref[...] * sh_ref[...] + th_ref[...]
        b = f_ref[...] * sf_ref[...] + tf_ref[...]
        o_ref[...] = (
            jnp.dot(a, wh_ref[...], preferred_element_type=jnp.float32)
            + jnp.dot(b, wf_ref[...], preferred_element_type=jnp.float32)
            + cb_ref[...])

    return pl.pallas_call(
        body,
        grid=(nblk,),
        in_specs=[
            pl.BlockSpec((RB, CH), lambda i: (i, 0)),
            pl.BlockSpec((RB, CF), lambda i: (i, 0)),
            pl.BlockSpec((CH, _OUT), lambda i: (0, 0)),
            pl.BlockSpec((CF, _OUT), lambda i: (0, 0)),
            pl.BlockSpec((1, CH), lambda i: (0, 0)),
            pl.BlockSpec((1, CH), lambda i: (0, 0)),
            pl.BlockSpec((1, CF), lambda i: (0, 0)),
            pl.BlockSpec((1, CF), lambda i: (0, 0)),
            pl.BlockSpec((1, _OUT), lambda i: (0, 0)),
        ],
        out_specs=pl.BlockSpec((RB, _OUT), lambda i: (i, 0)),
        out_shape=jax.ShapeDtypeStruct((_BM, _OUT), jnp.float32),
        interpret=interpret,
    )(hm2, fm2, Wh, Wf, sh, th, sf, tf, cb)


def kernel(points, queries, feat_prev, bn1_g, bn1_b, W1, b1, bn2_g, bn2_b,
           W2, b2, bn3_g, bn3_b, convW, convb):
    f32 = jnp.float32
    pts = points.reshape(_B, _N, 3)
    q2 = queries.reshape(_B, _M, 3)
    fp2d = feat_prev.reshape(_B * _N, _FPREV)
    ptsT = pts.transpose(0, 2, 1)

    idxf = _knn_topk(q2, ptsT)
    idx_flat = idxf.reshape(_ROWS)
    idx2d = idxf.reshape(_ROWS // 128, 128)

    pf3 = pts.reshape(_B * _N, 3)
    px, py, pz = pf3[:, 0], pf3[:, 1], pf3[:, 2]
    qf3 = q2.reshape(_BM, 3)
    qxr = jnp.repeat(qf3[:, 0], _K)
    qyr = jnp.repeat(qf3[:, 1], _K)
    qzr = jnp.repeat(qf3[:, 2], _K)
    lx, ly, lz, fmax, st1 = _sc_gather(px, py, pz, qxr, qyr, qzr,
                                       idx_flat, idx2d, fp2d)

    cnt1 = f32(_ROWS)
    st1r = st1.reshape(_NW, 8, _LN)
    s1sum = jnp.sum(st1r[:, 0:3, :], axis=(0, 2))
    s1sq = jnp.sum(st1r[:, 3:6, :], axis=(0, 2))
    mu1 = s1sum / cnt1
    var1 = s1sq / cnt1 - mu1 * mu1
    inv1 = lax.rsqrt(var1 + _EPS)
    sc1 = bn1_g * inv1
    sh1 = bn1_b - mu1 * sc1
    W1eff = W1 * sc1[None, :]
    b1eff = b1 + W1 @ sh1

    X = jnp.stack([lx, ly, lz], axis=-1)
    r1, p3 = _mlp1(X, W1eff.T, b1eff.reshape(1, _PF // 2))

    mu2 = jnp.sum(p3[:, 0, :], axis=0) / cnt1
    var2 = jnp.sum(p3[:, 1, :], axis=0) / cnt1 - mu2 * mu2
    inv2 = lax.rsqrt(var2 + _EPS)
    sc2 = bn2_g * inv2
    sh2 = bn2_b - mu2 * sc2
    W2eff = W2 * sc2[None, :]
    b2eff = b2 + W2 @ sh2

    hmax, ph, pf = _mlp2pool(r1, fmax, W2eff.T, b2eff.reshape(1, _PF))

    cnt3 = f32(_BM * _DIV)
    mu3h = jnp.sum(ph[:, 0, :], axis=0) / cnt3
    var3h = jnp.sum(ph[:, 1, :], axis=0) / cnt3 - mu3h * mu3h
    mu3f = jnp.sum(pf[:, 0, :], axis=0) / cnt3
    var3f = jnp.sum(pf[:, 1, :], axis=0) / cnt3 - mu3f * mu3f
    inv3h = lax.rsqrt(var3h + _EPS)
    inv3f = lax.rsqrt(var3f + _EPS)
    sch = bn3_g[:_PF] * inv3h
    thv = bn3_b[:_PF] - mu3h * sch
    scf = bn3_g[_PF:] * inv3f
    tfv = bn3_b[_PF:] - mu3f * scf

    Wh = convW[:, :_PF, :].transpose(2, 1, 0).reshape(_DIV * _PF, _OUT)
    Wf = convW[:, _PF:, :].transpose(2, 1, 0).reshape(_DIV * _FPREV, _OUT)
    sh256 = jnp.tile(sch, _DIV).reshape(1, _DIV * _PF)
    th256 = jnp.tile(thv, _DIV).reshape(1, _DIV * _PF)
    sf512 = jnp.tile(scf, _DIV).reshape(1, _DIV * _FPREV)
    tf512 = jnp.tile(tfv, _DIV).reshape(1, _DIV * _FPREV)
    hm2 = hmax.reshape(_BM, _DIV * _PF)
    fm2 = fmax.reshape(_BM, _DIV * _FPREV)
    out = _conv(hm2, fm2, Wh, Wf, sh256, th256, sf512, tf512,
                convb.reshape(1, _OUT))
    return out.reshape(_B, _L, _M, _OUT)
